# baseline (speedup 1.0000x reference)
import jax
import jax.numpy as jnp
from jax.experimental import pallas as pl
from jax.experimental.pallas import tpu as pltpu

_SIZE = 65536
_DIM = 128
_BATCH = 1024
_NCH = 2
_RB = _DIM // _NCH


def _body(out_hbm, bank_hbm, oo_hbm, snap_hbm, new_hbm, *scratch):
    bufs = scratch[:_NCH]
    vin, vout, isem, ssem, nsem, osem = scratch[_NCH:]

    def in_cp(j):
        return pltpu.make_async_copy(
            bank_hbm.at[pl.ds(j * _RB, _RB), :], bufs[j], isem.at[j])

    def snap_cp(j):
        return pltpu.make_async_copy(
            bufs[j], snap_hbm.at[pl.ds(j * _RB, _RB), :], ssem.at[j])

    def new_cp(j):
        return pltpu.make_async_copy(
            bufs[j].at[:, pl.ds(_BATCH, _SIZE - _BATCH)],
            new_hbm.at[pl.ds(j * _RB, _RB), pl.ds(_BATCH, _SIZE - _BATCH)],
            nsem.at[j])

    def t_cp(j):
        return pltpu.make_async_copy(
            vout.at[pl.ds(j * _RB, _RB), :],
            new_hbm.at[pl.ds(j * _RB, _RB), pl.ds(0, _BATCH)],
            osem.at[1 + j])

    ocp_in = pltpu.make_async_copy(out_hbm, vin, osem.at[0])
    ocp_in.start()
    in_cp(0).start()
    ocp_in.wait()
    vout[...] = jnp.transpose(vin[...])
    oo_cp = pltpu.make_async_copy(vin, oo_hbm, osem.at[_NCH + 1])
    oo_cp.start()
    for j in range(_NCH):
        t_cp(j).start()
    for j in range(_NCH):
        in_cp(j).wait()
        snap_cp(j).start()
        new_cp(j).start()
        if j + 1 < _NCH:
            in_cp(j + 1).start()
    for j in range(_NCH):
        snap_cp(j).wait()
        new_cp(j).wait()
        t_cp(j).wait()
    oo_cp.wait()


def kernel(output, bank):
    out_shapes = (
        jax.ShapeDtypeStruct((_BATCH, _DIM), output.dtype),
        jax.ShapeDtypeStruct((_DIM, _SIZE), bank.dtype),
        jax.ShapeDtypeStruct((_DIM, _SIZE), bank.dtype),
    )
    out, snap, new = pl.pallas_call(
        _body,
        in_specs=[
            pl.BlockSpec(memory_space=pl.ANY),
            pl.BlockSpec(memory_space=pl.ANY),
        ],
        out_specs=[
            pl.BlockSpec(memory_space=pl.ANY),
            pl.BlockSpec(memory_space=pl.ANY),
            pl.BlockSpec(memory_space=pl.ANY),
        ],
        out_shape=out_shapes,
        scratch_shapes=(
            [pltpu.VMEM((_RB, _SIZE), jnp.float32) for _ in range(_NCH)]
            + [
                pltpu.VMEM((_BATCH, _DIM), jnp.float32),
                pltpu.VMEM((_DIM, _BATCH), jnp.float32),
                pltpu.SemaphoreType.DMA((_NCH,)),
                pltpu.SemaphoreType.DMA((_NCH,)),
                pltpu.SemaphoreType.DMA((_NCH,)),
                pltpu.SemaphoreType.DMA((_NCH + 2,)),
            ]
        ),
    )(output, bank)
    return (out, snap, new)


# speedup vs baseline: 1.0541x; 1.0541x over previous
import jax
import jax.numpy as jnp
from jax.experimental import pallas as pl
from jax.experimental.pallas import tpu as pltpu

_SIZE = 65536
_DIM = 128
_BATCH = 1024
_NCH = 2
_CH = _SIZE // _NCH


def _body(out_hbm, bank_hbm, oo_hbm, snap_hbm, new_hbm, *scratch):
    bufs = scratch[:_NCH]
    vin, vout, isem, ssem, nsem, osem = scratch[_NCH:]

    def in_cp(j):
        return pltpu.make_async_copy(
            bank_hbm.at[:, pl.ds(j * _CH, _CH)], bufs[j], isem.at[j])

    def snap_cp(j):
        return pltpu.make_async_copy(
            bufs[j], snap_hbm.at[:, pl.ds(j * _CH, _CH)], ssem.at[j])

    def new_cp(j):
        if j == 0:
            return pltpu.make_async_copy(
                bufs[0].at[:, pl.ds(_BATCH, _CH - _BATCH)],
                new_hbm.at[:, pl.ds(_BATCH, _CH - _BATCH)], nsem.at[0])
        return pltpu.make_async_copy(
            bufs[j], new_hbm.at[:, pl.ds(j * _CH, _CH)], nsem.at[j])

    ocp_in = pltpu.make_async_copy(out_hbm, vin, osem.at[0])
    ocp_in.start()
    for j in range(_NCH):
        in_cp(j).start()
    ocp_in.wait()
    vout[...] = jnp.transpose(vin[...])
    oo_cp = pltpu.make_async_copy(vin, oo_hbm, osem.at[1])
    oo_cp.start()
    t_cp = pltpu.make_async_copy(vout, new_hbm.at[:, pl.ds(0, _BATCH)], osem.at[2])
    t_cp.start()
    for j in range(_NCH):
        in_cp(j).wait()
        snap_cp(j).start()
        new_cp(j).start()
    for j in range(_NCH):
        snap_cp(j).wait()
        new_cp(j).wait()
    oo_cp.wait()
    t_cp.wait()


def kernel(output, bank):
    out_shapes = (
        jax.ShapeDtypeStruct((_BATCH, _DIM), output.dtype),
        jax.ShapeDtypeStruct((_DIM, _SIZE), bank.dtype),
        jax.ShapeDtypeStruct((_DIM, _SIZE), bank.dtype),
    )
    out, snap, new = pl.pallas_call(
        _body,
        in_specs=[
            pl.BlockSpec(memory_space=pl.ANY),
            pl.BlockSpec(memory_space=pl.ANY),
        ],
        out_specs=[
            pl.BlockSpec(memory_space=pl.ANY),
            pl.BlockSpec(memory_space=pl.ANY),
            pl.BlockSpec(memory_space=pl.ANY),
        ],
        out_shape=out_shapes,
        scratch_shapes=(
            [pltpu.VMEM((_DIM, _CH), jnp.float32) for _ in range(_NCH)]
            + [
                pltpu.VMEM((_BATCH, _DIM), jnp.float32),
                pltpu.VMEM((_DIM, _BATCH), jnp.float32),
                pltpu.SemaphoreType.DMA((_NCH,)),
                pltpu.SemaphoreType.DMA((_NCH,)),
                pltpu.SemaphoreType.DMA((_NCH,)),
                pltpu.SemaphoreType.DMA((3,)),
            ]
        ),
    )(output, bank)
    return (out, snap, new)
